# speedup vs baseline: 3.0384x; 3.0384x over previous
import functools

import jax
import jax.numpy as jnp
from jax import lax
from jax.experimental import pallas as pl
from jax.experimental.pallas import tpu as pltpu
from jax.experimental.pallas import tpu_sc as plsc

N = 10000
E = 320000
F = 128

NC = 2
NS = 16
NW = NC * NS
CHUNK = 128
EDGES_PER_W = 10240
NCHUNK = EDGES_PER_W // CHUNK
E_PAD = NW * EDGES_PER_W
N_PAD = 10240
ROWS_PER_TILE = N_PAD // NS
BLKS_PER_TILE = ROWS_PER_TILE // CHUNK
DUMMY_DST = N


def _sc_agg_body(src_hbm, dst_hbm, x_hbm, out_hbm,
                 src_v, dst_v, rows_v, agg_sh, sem):
    c = lax.axis_index("c")
    s = lax.axis_index("s")
    wid = s * NC + c
    tid = s

    def _zero_row(i, carry):
        for j in range(F // 16):
            rows_v[i, pl.ds(j * 16, 16)] = jnp.zeros((16,), jnp.float32)
        return carry

    lax.fori_loop(0, CHUNK, _zero_row, 0)

    def _zero_blk(b, carry):
        pltpu.sync_copy(rows_v, agg_sh.at[pl.ds(tid * ROWS_PER_TILE + b * CHUNK, CHUNK)])
        return carry

    lax.fori_loop(0, BLKS_PER_TILE, _zero_blk, 0)
    plsc.subcore_barrier()

    pltpu.sync_copy(src_hbm.at[wid], src_v)
    pltpu.sync_copy(dst_hbm.at[wid], dst_v)

    def _edge_chunk(j, carry):
        pltpu.async_copy(x_hbm.at[src_v.at[j]], rows_v, sem).wait()
        pltpu.sync_copy(rows_v, agg_sh.at[dst_v.at[j]], add=True)
        return carry

    lax.fori_loop(0, NCHUNK, _edge_chunk, 0)
    plsc.subcore_barrier()

    def _writeback(b, carry):
        base = tid * ROWS_PER_TILE + b * CHUNK
        pltpu.sync_copy(agg_sh.at[pl.ds(base, CHUNK)], rows_v)
        pltpu.sync_copy(rows_v, out_hbm.at[pl.ds(c * N_PAD + base, CHUNK)])
        return carry

    lax.fori_loop(0, BLKS_PER_TILE, _writeback, 0)


_sc_agg = pl.kernel(
    _sc_agg_body,
    out_type=jax.ShapeDtypeStruct((NC * N_PAD, F), jnp.float32),
    mesh=plsc.VectorSubcoreMesh(core_axis_name="c", subcore_axis_name="s"),
    scratch_types=[
        pltpu.VMEM((NCHUNK, CHUNK), jnp.int32),
        pltpu.VMEM((NCHUNK, CHUNK), jnp.int32),
        pltpu.VMEM((CHUNK, F), jnp.float32),
        pltpu.VMEM_SHARED((N_PAD, F), jnp.float32),
        pltpu.SemaphoreType.DMA,
    ],
)


def _mlp_body(p0, p1, x, eps, w1t, b1, g1, be1, w2t, b2, g2, be2, out):
    agg = p0[...] + p1[...] + eps[...] * x[...]
    h = jnp.dot(agg, w1t[...], preferred_element_type=jnp.float32) + b1[...]
    mu = jnp.mean(h, axis=0, keepdims=True)
    var = jnp.mean((h - mu) ** 2, axis=0, keepdims=True)
    h = (h - mu) * lax.rsqrt(var + 1e-5) * g1[...] + be1[...]
    h = jnp.where(h > 0, h, jnp.exp(h) - 1.0)
    h = jnp.dot(h, w2t[...], preferred_element_type=jnp.float32) + b2[...]
    mu = jnp.mean(h, axis=0, keepdims=True)
    var = jnp.mean((h - mu) ** 2, axis=0, keepdims=True)
    h = (h - mu) * lax.rsqrt(var + 1e-5) * g2[...] + be2[...]
    out[...] = jnp.where(h > 0, h, jnp.exp(h) - 1.0)


_mlp = pl.pallas_call(
    _mlp_body,
    out_shape=jax.ShapeDtypeStruct((N, F), jnp.float32),
)


def kernel(x, edge_index, epsilon, W1, b1, g1, beta1, W2, b2, g2, beta2):
    dst = edge_index[0]
    src = edge_index[1]
    pad = E_PAD - E
    src_p = jnp.concatenate([src, jnp.zeros((pad,), jnp.int32)]).reshape(NW, NCHUNK, CHUNK)
    dst_p = jnp.concatenate([dst, jnp.full((pad,), DUMMY_DST, jnp.int32)]).reshape(NW, NCHUNK, CHUNK)
    parts = _sc_agg(src_p, dst_p, x)
    p0 = parts[:N]
    p1 = parts[N_PAD:N_PAD + N]
    return _mlp(p0, p1, x, epsilon,
                W1.T, b1.reshape(1, F), g1.reshape(1, F), beta1.reshape(1, F),
                W2.T, b2.reshape(1, F), g2.reshape(1, F), beta2.reshape(1, F))
